# Initial kernel scaffold; baseline (speedup 1.0000x reference)
import jax
import jax.numpy as jnp
from jax.experimental import pallas as pl


def kernel(feat, edge_index, weight1, weight2):
    raise NotImplementedError("write your pallas kernel here")


# speedup vs baseline: 11.3771x; 11.3771x over previous
import jax
import jax.numpy as jnp
from jax import lax
from jax.experimental import pallas as pl
from jax.experimental.pallas import tpu as pltpu
from jax.experimental.pallas import tpu_sc as plsc

N_NODES = 10000
D = 128
N_EDGES = 320000

NC = 2
NS = 16
NW = NC * NS
EPW = N_EDGES // NW
C = 100
Q = 5
SCH = 20
SEG = 640
LAST_SEG = N_NODES - 15 * SEG
ZR = 80

_sc_mesh = plsc.VectorSubcoreMesh(core_axis_name="c", subcore_axis_name="s")


def _agg_body(src_hbm, dst_hbm, feat_hbm, zeros_hbm, out_hbm,
              sidx, didx, rows0, rows1, acc, sem0, sem1):
    cid = lax.axis_index("c")
    sid = lax.axis_index("s")
    wid = sid * NC + cid

    pltpu.sync_copy(zeros_hbm, rows0)

    @pl.when(sid < NS - 1)
    def _():
        for k in range(SEG // ZR):
            pltpu.sync_copy(rows0.at[pl.ds(0, ZR)],
                            acc.at[pl.ds(sid * SEG + k * ZR, ZR)])

    @pl.when(sid == NS - 1)
    def _():
        for k in range(LAST_SEG // ZR):
            pltpu.sync_copy(rows0.at[pl.ds(0, ZR)],
                            acc.at[pl.ds(15 * SEG + k * ZR, ZR)])

    plsc.subcore_barrier()

    def _gather(c, rbuf, sem):
        pltpu.async_copy(feat_hbm.at[sidx.at[c]], rbuf, sem)

    def _gwait(rbuf, sem):
        pltpu.make_async_copy(feat_hbm.at[sidx.at[0]], rbuf, sem).wait()

    for q in range(Q):
        pltpu.sync_copy(src_hbm.at[wid, q], sidx)
        pltpu.sync_copy(dst_hbm.at[wid, q], didx)

        _gather(0, rows0, sem0)
        _gather(1, rows1, sem1)

        def _pair(p, carry):
            c0 = 2 * p
            _gwait(rows0, sem0)
            pltpu.sync_copy(rows0, acc.at[didx.at[c0]], add=True)

            @pl.when(c0 + 2 < SCH)
            def _():
                _gather(c0 + 2, rows0, sem0)

            c1 = c0 + 1
            _gwait(rows1, sem1)
            pltpu.sync_copy(rows1, acc.at[didx.at[c1]], add=True)

            @pl.when(c1 + 2 < SCH)
            def _():
                _gather(c1 + 2, rows1, sem1)

            return carry

        lax.fori_loop(0, SCH // 2, _pair, 0)

    plsc.subcore_barrier()

    @pl.when(sid < NS - 1)
    def _():
        pltpu.sync_copy(acc.at[pl.ds(sid * SEG, SEG)],
                        out_hbm.at[cid, pl.ds(sid * SEG, SEG)])

    @pl.when(sid == NS - 1)
    def _():
        pltpu.sync_copy(acc.at[pl.ds(15 * SEG, LAST_SEG)],
                        out_hbm.at[cid, pl.ds(15 * SEG, LAST_SEG)])


_agg = pl.kernel(
    _agg_body,
    out_type=jax.ShapeDtypeStruct((NC, N_NODES, D), jnp.float32),
    mesh=_sc_mesh,
    scratch_types=[
        pltpu.VMEM((SCH, C), jnp.int32),
        pltpu.VMEM((SCH, C), jnp.int32),
        pltpu.VMEM((C, D), jnp.float32),
        pltpu.VMEM((C, D), jnp.float32),
        pltpu.VMEM_SHARED((N_NODES, D), jnp.float32),
        pltpu.SemaphoreType.DMA,
        pltpu.SemaphoreType.DMA,
    ],
)


def _mm_body(feat_ref, p_ref, w1_ref, w2_ref, o_ref):
    agg = p_ref[0] + p_ref[1]
    o_ref[...] = (
        jnp.dot(feat_ref[...], w1_ref[...], preferred_element_type=jnp.float32)
        + jnp.dot(agg, w2_ref[...], preferred_element_type=jnp.float32)
    )


_ROWS_BLK = 1000


def _mm(feat, partials, w1, w2):
    return pl.pallas_call(
        _mm_body,
        grid=(N_NODES // _ROWS_BLK,),
        in_specs=[
            pl.BlockSpec((_ROWS_BLK, D), lambda i: (i, 0)),
            pl.BlockSpec((NC, _ROWS_BLK, D), lambda i: (0, i, 0)),
            pl.BlockSpec((D, D), lambda i: (0, 0)),
            pl.BlockSpec((D, D), lambda i: (0, 0)),
        ],
        out_specs=pl.BlockSpec((_ROWS_BLK, D), lambda i: (i, 0)),
        out_shape=jax.ShapeDtypeStruct((N_NODES, D), jnp.float32),
    )(feat, partials, w1, w2)


@jax.jit
def kernel(feat, edge_index, weight1, weight2):
    src4 = edge_index[0].reshape(NW, Q, SCH, C)
    dst4 = edge_index[1].reshape(NW, Q, SCH, C)
    zeros = jnp.zeros((C, D), jnp.float32)
    partials = _agg(src4, dst4, feat, zeros)
    return _mm(feat, partials, weight1, weight2)
